# baseline (speedup 1.0000x reference)
import functools

import jax
import jax.numpy as jnp
from jax import lax
from jax.experimental import pallas as pl
from jax.experimental.pallas import tpu as pltpu
from jax.experimental.pallas import tpu_sc as plsc

B = 16384
FACT = 64
IMGF = 512

_info = plsc.get_sparse_core_info()
NC = _info.num_cores
NS = _info.num_subcores
NW = NC * NS
BPW = B // NW
CH = 64
NCH = BPW // CH


def _sc_gather(user2, item2, Gu, Gi, L, F):
    mesh = plsc.VectorSubcoreMesh(core_axis_name="c", subcore_axis_name="s")

    @functools.partial(
        pl.kernel, mesh=mesh,
        out_type=[
            jax.ShapeDtypeStruct((B, FACT), jnp.float32),
            jax.ShapeDtypeStruct((B, FACT), jnp.float32),
            jax.ShapeDtypeStruct((B, FACT), jnp.float32),
            jax.ShapeDtypeStruct((B, IMGF), jnp.float32),
        ],
        scratch_types=[
            pltpu.VMEM((NCH, CH), jnp.int32),
            pltpu.VMEM((NCH, CH), jnp.int32),
            pltpu.VMEM((CH, FACT), jnp.float32),
            pltpu.VMEM((CH, FACT), jnp.float32),
            pltpu.VMEM((CH, FACT), jnp.float32),
            pltpu.VMEM((CH, IMGF), jnp.float32),
            pltpu.SemaphoreType.DMA,
        ],
        compiler_params=pltpu.CompilerParams(use_tc_tiling_on_sc=False),
    )
    def k(user_h, item_h, gu_h, gi_h, l_h, f_h,
          gu_o, gi_o, li_o, fi_o,
          uidx, iidx, gub, gib, lb, fb, sem):
        wid = lax.axis_index("s") * NC + lax.axis_index("c")
        pltpu.sync_copy(user_h.at[wid], uidx)
        pltpu.sync_copy(item_h.at[wid], iidx)
        base = wid * BPW
        for c in range(NCH):
            off = base + c * CH
            cu = pltpu.async_copy(gu_h.at[uidx.at[c]], gub, sem)
            ci = pltpu.async_copy(gi_h.at[iidx.at[c]], gib, sem)
            cl = pltpu.async_copy(l_h.at[iidx.at[c]], lb, sem)
            cf = pltpu.async_copy(f_h.at[iidx.at[c]], fb, sem)
            cu.wait()
            pltpu.sync_copy(gub, gu_o.at[pl.ds(off, CH)])
            ci.wait()
            pltpu.sync_copy(gib, gi_o.at[pl.ds(off, CH)])
            cl.wait()
            pltpu.sync_copy(lb, li_o.at[pl.ds(off, CH)])
            cf.wait()
            pltpu.sync_copy(fb, fi_o.at[pl.ds(off, CH)])

    return k(user2, item2, Gu, Gi, L, F)


def _tc_combine(feature_i, gamma_u, gamma_i, l_i, E):
    BB = 2048

    def body(fe, gu, gi, li, e, out):
        t = jnp.dot(fe[...], e[...], preferred_element_type=jnp.float32)
        out[...] = jnp.sum(gu[...] * (t - li[...] + gi[...]), axis=1,
                           keepdims=True)

    return pl.pallas_call(
        body,
        grid=(B // BB,),
        in_specs=[
            pl.BlockSpec((BB, IMGF), lambda i: (i, 0)),
            pl.BlockSpec((BB, FACT), lambda i: (i, 0)),
            pl.BlockSpec((BB, FACT), lambda i: (i, 0)),
            pl.BlockSpec((BB, FACT), lambda i: (i, 0)),
            pl.BlockSpec((IMGF, FACT), lambda i: (0, 0)),
        ],
        out_specs=pl.BlockSpec((BB, 1), lambda i: (i, 0)),
        out_shape=jax.ShapeDtypeStruct((B, 1), jnp.float32),
    )(feature_i, gamma_u, gamma_i, l_i, E)


def kernel(user, item, Gu, Gi, L, E, F):
    user2 = user.reshape(NW, NCH, CH)
    item2 = item.reshape(NW, NCH, CH)
    gamma_u, gamma_i, l_i, feature_i = _sc_gather(user2, item2, Gu, Gi, L, F)
    xui = _tc_combine(feature_i, gamma_u, gamma_i, l_i, E).reshape(B)
    return (xui, gamma_u, gamma_i, feature_i, l_i)


# speedup vs baseline: 1.1717x; 1.1717x over previous
import functools

import jax
import jax.numpy as jnp
from jax import lax
from jax.experimental import pallas as pl
from jax.experimental.pallas import tpu as pltpu
from jax.experimental.pallas import tpu_sc as plsc

B = 16384
FACT = 64
IMGF = 512

_info = plsc.get_sparse_core_info()
NC = _info.num_cores
NS = _info.num_subcores
NW = NC * NS
BPW = B // NW
CH = 64
NCH = BPW // CH

_MESH = plsc.VectorSubcoreMesh(core_axis_name="c", subcore_axis_name="s")


def _sc_gather_f(item2, F):
    @functools.partial(
        pl.kernel, mesh=_MESH,
        out_type=jax.ShapeDtypeStruct((B, IMGF), jnp.float32),
        scratch_types=[
            pltpu.VMEM((NCH, CH), jnp.int32),
            pltpu.VMEM((2, CH, IMGF), jnp.float32),
            pltpu.SemaphoreType.DMA,
            pltpu.SemaphoreType.DMA,
        ],
    )
    def k(item_h, f_h, fi_o, iidx, fb, gsem, wsem):
        wid = lax.axis_index("s") * NC + lax.axis_index("c")
        base = wid * BPW
        pltpu.sync_copy(item_h.at[wid], iidx)
        h_g = [None, None]
        h_w = [None, None]
        for c in range(NCH):
            s = c % 2
            if h_w[s] is not None:
                h_w[s].wait()
            h_g[s] = pltpu.async_copy(f_h.at[iidx.at[c]], fb.at[s], gsem)
            if c > 0:
                p = 1 - s
                h_g[p].wait()
                h_w[p] = pltpu.async_copy(
                    fb.at[p], fi_o.at[pl.ds(base + (c - 1) * CH, CH)], wsem)
        last = (NCH - 1) % 2
        h_g[last].wait()
        h_w[last] = pltpu.async_copy(
            fb.at[last], fi_o.at[pl.ds(base + (NCH - 1) * CH, CH)], wsem)
        h_w[1 - last].wait()
        h_w[last].wait()

    return k(item2, F)


def _sc_gather_small(user2, item2, Gu, Gi, L):
    @functools.partial(
        pl.kernel, mesh=_MESH,
        out_type=[
            jax.ShapeDtypeStruct((B, FACT), jnp.float32),
            jax.ShapeDtypeStruct((B, FACT), jnp.float32),
            jax.ShapeDtypeStruct((B, FACT), jnp.float32),
        ],
        scratch_types=[
            pltpu.VMEM((NCH, CH), jnp.int32),
            pltpu.VMEM((NCH, CH), jnp.int32),
            pltpu.VMEM((2, CH, FACT), jnp.float32),
            pltpu.VMEM((2, CH, FACT), jnp.float32),
            pltpu.VMEM((2, CH, FACT), jnp.float32),
            pltpu.SemaphoreType.DMA,
            pltpu.SemaphoreType.DMA,
        ],
        compiler_params=pltpu.CompilerParams(use_tc_tiling_on_sc=False),
    )
    def k(user_h, item_h, gu_h, gi_h, l_h,
          gu_o, gi_o, li_o, uidx, iidx, gub, gib, lb, gsem, wsem):
        wid = lax.axis_index("s") * NC + lax.axis_index("c")
        base = wid * BPW
        pltpu.sync_copy(user_h.at[wid], uidx)
        pltpu.sync_copy(item_h.at[wid], iidx)
        bufs = (gub, gib, lb)
        h_g = [None, None]
        h_w = [None, None]
        for c in range(NCH):
            s = c % 2
            if h_w[s] is not None:
                for h in h_w[s]:
                    h.wait()
            h_g[s] = (
                pltpu.async_copy(gu_h.at[uidx.at[c]], gub.at[s], gsem),
                pltpu.async_copy(gi_h.at[iidx.at[c]], gib.at[s], gsem),
                pltpu.async_copy(l_h.at[iidx.at[c]], lb.at[s], gsem),
            )
            if c > 0:
                p = 1 - s
                off = base + (c - 1) * CH
                for h in h_g[p]:
                    h.wait()
                h_w[p] = tuple(
                    pltpu.async_copy(bf.at[p], o.at[pl.ds(off, CH)], wsem)
                    for bf, o in zip(bufs, (gu_o, gi_o, li_o)))
        last = (NCH - 1) % 2
        off = base + (NCH - 1) * CH
        for h in h_g[last]:
            h.wait()
        h_w[last] = tuple(
            pltpu.async_copy(bf.at[last], o.at[pl.ds(off, CH)], wsem)
            for bf, o in zip(bufs, (gu_o, gi_o, li_o)))
        for h in h_w[1 - last]:
            h.wait()
        for h in h_w[last]:
            h.wait()

    return k(user2, item2, Gu, Gi, L)


def _tc_combine(feature_i, gamma_u, gamma_i, l_i, E):
    BB = 2048

    def body(fe, gu, gi, li, e, out):
        t = jnp.dot(fe[...], e[...], preferred_element_type=jnp.float32)
        out[...] = jnp.sum(gu[...] * (t - li[...] + gi[...]), axis=1,
                           keepdims=True)

    return pl.pallas_call(
        body,
        grid=(B // BB,),
        in_specs=[
            pl.BlockSpec((BB, IMGF), lambda i: (i, 0)),
            pl.BlockSpec((BB, FACT), lambda i: (i, 0)),
            pl.BlockSpec((BB, FACT), lambda i: (i, 0)),
            pl.BlockSpec((BB, FACT), lambda i: (i, 0)),
            pl.BlockSpec((IMGF, FACT), lambda i: (0, 0)),
        ],
        out_specs=pl.BlockSpec((BB, 1), lambda i: (i, 0)),
        out_shape=jax.ShapeDtypeStruct((B, 1), jnp.float32),
    )(feature_i, gamma_u, gamma_i, l_i, E)


def kernel(user, item, Gu, Gi, L, E, F):
    user2 = user.reshape(NW, NCH, CH)
    item2 = item.reshape(NW, NCH, CH)
    feature_i = _sc_gather_f(item2, F)
    gamma_u, gamma_i, l_i = _sc_gather_small(user2, item2, Gu, Gi, L)
    xui = _tc_combine(feature_i, gamma_u, gamma_i, l_i, E).reshape(B)
    return (xui, gamma_u, gamma_i, feature_i, l_i)
